# baseline (speedup 1.0000x reference)
from functools import partial

import jax
import jax.numpy as jnp
from jax.experimental import pallas as pl
from jax.experimental.pallas import tpu as pltpu


def _gcn_body(x_ref, a_ref, w_ref, b_ref, o_ref, *, T, V, Cout):
    a32 = a_ref[0]
    d = jnp.sum(a32, axis=-1, keepdims=True)
    dinv = jax.lax.rsqrt(d)
    a16 = a32.astype(jnp.bfloat16)

    x = x_ref[0].astype(jnp.bfloat16)
    proj = jnp.dot(x, w_ref[...],
                   preferred_element_type=jnp.float32)

    s = jnp.concatenate(
        [(dinv * proj[t * V:(t + 1) * V]).astype(jnp.bfloat16)
         for t in range(T)], axis=1)

    agg = jnp.dot(a16, s,
                  preferred_element_type=jnp.float32)

    bias = b_ref[...]
    for t in range(T):
        o_ref[0, t] = (dinv * agg[:, t * Cout:(t + 1) * Cout]
                       + bias).astype(o_ref.dtype)


def kernel(X, A, weight, bias):
    N, T, V, Cin = X.shape
    Cout = weight.shape[1]

    Xf = X.reshape(N, T * V, Cin)
    w16 = weight.astype(jnp.bfloat16)
    bias2 = bias.reshape(1, Cout)

    return pl.pallas_call(
        partial(_gcn_body, T=T, V=V, Cout=Cout),
        out_shape=jax.ShapeDtypeStruct((N, T, V, Cout), X.dtype),
        grid=(N,),
        in_specs=[
            pl.BlockSpec((1, T * V, Cin), lambda n: (n, 0, 0)),
            pl.BlockSpec((1, V, V), lambda n: (n, 0, 0)),
            pl.BlockSpec((Cin, Cout), lambda n: (0, 0)),
            pl.BlockSpec((1, Cout), lambda n: (0, 0)),
        ],
        out_specs=pl.BlockSpec((1, T, V, Cout), lambda n: (n, 0, 0, 0)),
        compiler_params=pltpu.CompilerParams(
            dimension_semantics=("parallel",)),
    )(Xf, A, w16, bias2)


# speedup vs baseline: 1.3121x; 1.3121x over previous
from functools import partial

import jax
import jax.numpy as jnp
from jax.experimental import pallas as pl
from jax.experimental.pallas import tpu as pltpu


def _gcn_body(x_ref, a_ref, w_ref, b_ref, o_ref, *, T, V, Cout):
    a32 = a_ref[0]
    d = jnp.sum(a32, axis=-1, keepdims=True)
    dinv = jax.lax.rsqrt(d)
    a16 = a32.astype(jnp.bfloat16)

    TV, Cin = T * V, x_ref.shape[-1]
    x = x_ref[0].reshape(TV, Cin).astype(jnp.bfloat16)
    proj = jnp.dot(x, w_ref[...],
                   preferred_element_type=jnp.float32)

    s = jnp.concatenate(
        [(dinv * proj[t * V:(t + 1) * V]).astype(jnp.bfloat16)
         for t in range(T)], axis=1)

    agg = jnp.dot(a16, s,
                  preferred_element_type=jnp.float32)

    bias = b_ref[...]
    for t in range(T):
        o_ref[0, t] = (dinv * agg[:, t * Cout:(t + 1) * Cout]
                       + bias).astype(o_ref.dtype)


def kernel(X, A, weight, bias):
    N, T, V, Cin = X.shape
    Cout = weight.shape[1]

    w16 = weight.astype(jnp.bfloat16)
    bias2 = bias.reshape(1, Cout)

    return pl.pallas_call(
        partial(_gcn_body, T=T, V=V, Cout=Cout),
        out_shape=jax.ShapeDtypeStruct((N, T, V, Cout), X.dtype),
        grid=(N,),
        in_specs=[
            pl.BlockSpec((1, T, V, Cin), lambda n: (n, 0, 0, 0)),
            pl.BlockSpec((1, V, V), lambda n: (n, 0, 0)),
            pl.BlockSpec((Cin, Cout), lambda n: (0, 0)),
            pl.BlockSpec((1, Cout), lambda n: (0, 0)),
        ],
        out_specs=pl.BlockSpec((1, T, V, Cout), lambda n: (n, 0, 0, 0)),
        compiler_params=pltpu.CompilerParams(
            dimension_semantics=("parallel",)),
    )(X, A, w16, bias2)


# speedup vs baseline: 1.4655x; 1.1169x over previous
from functools import partial

import jax
import jax.numpy as jnp
from jax.experimental import pallas as pl
from jax.experimental.pallas import tpu as pltpu


def _gcn_body(x_ref, a_ref, w_ref, b_ref, o_ref, *, nb, T, V, Cout):
    TV, Cin = T * V, x_ref.shape[-1]
    bias = b_ref[...]
    for b in range(nb):
        a32 = a_ref[b]
        d = jnp.sum(a32, axis=-1, keepdims=True)
        dinv = jax.lax.rsqrt(d)
        a16 = a32.astype(jnp.bfloat16)

        x = x_ref[b].reshape(TV, Cin).astype(jnp.bfloat16)
        proj = jnp.dot(x, w_ref[...],
                       preferred_element_type=jnp.float32)

        s = jnp.concatenate(
            [(dinv * proj[t * V:(t + 1) * V]).astype(jnp.bfloat16)
             for t in range(T)], axis=1)

        agg = jnp.dot(a16, s,
                      preferred_element_type=jnp.float32)

        for t in range(T):
            o_ref[b, t] = (dinv * agg[:, t * Cout:(t + 1) * Cout]
                           + bias).astype(o_ref.dtype)


def kernel(X, A, weight, bias):
    N, T, V, Cin = X.shape
    Cout = weight.shape[1]

    w16 = weight.astype(jnp.bfloat16)
    bias2 = bias.reshape(1, Cout)

    nb = next((c for c in (2, 1) if N % c == 0), 1)
    G = N // nb

    return pl.pallas_call(
        partial(_gcn_body, nb=nb, T=T, V=V, Cout=Cout),
        out_shape=jax.ShapeDtypeStruct((N, T, V, Cout), X.dtype),
        grid=(G,),
        in_specs=[
            pl.BlockSpec((nb, T, V, Cin), lambda n: (n, 0, 0, 0)),
            pl.BlockSpec((nb, V, V), lambda n: (n, 0, 0)),
            pl.BlockSpec((Cin, Cout), lambda n: (0, 0)),
            pl.BlockSpec((1, Cout), lambda n: (0, 0)),
        ],
        out_specs=pl.BlockSpec((nb, T, V, Cout), lambda n: (n, 0, 0, 0)),
        compiler_params=pltpu.CompilerParams(
            dimension_semantics=("parallel",)),
    )(X, A, w16, bias2)


# speedup vs baseline: 1.5127x; 1.0322x over previous
from functools import partial

import jax
import jax.numpy as jnp
from jax.experimental import pallas as pl
from jax.experimental.pallas import tpu as pltpu


def _gcn_body(x_ref, a_ref, w_ref, b_ref, o_ref, *, nb, T, V, Cout):
    TV, Cin = T * V, x_ref.shape[-1]
    bias = b_ref[...]
    for b in range(nb):
        a32 = a_ref[b]
        d = jnp.sum(a32, axis=-1, keepdims=True)
        dinv = jax.lax.rsqrt(d)
        a16 = a32.astype(jnp.bfloat16)

        x = x_ref[b].reshape(TV, Cin).astype(jnp.bfloat16)
        proj = jnp.dot(x, w_ref[...],
                       preferred_element_type=jnp.float32)

        s = jnp.concatenate(
            [(dinv * proj[t * V:(t + 1) * V]).astype(jnp.bfloat16)
             for t in range(T)], axis=1)

        agg = jnp.dot(a16, s,
                      preferred_element_type=jnp.float32)

        for t in range(T):
            o_ref[b, t] = (dinv * agg[:, t * Cout:(t + 1) * Cout]
                           + bias).astype(o_ref.dtype)


def kernel(X, A, weight, bias):
    N, T, V, Cin = X.shape
    Cout = weight.shape[1]

    w16 = weight.astype(jnp.bfloat16)
    bias2 = bias.reshape(1, Cout)

    nb = next((c for c in (4, 2, 1) if N % c == 0), 1)
    G = N // nb

    return pl.pallas_call(
        partial(_gcn_body, nb=nb, T=T, V=V, Cout=Cout),
        out_shape=jax.ShapeDtypeStruct((N, T, V, Cout), X.dtype),
        grid=(G,),
        in_specs=[
            pl.BlockSpec((nb, T, V, Cin), lambda n: (n, 0, 0, 0)),
            pl.BlockSpec((nb, V, V), lambda n: (n, 0, 0)),
            pl.BlockSpec((Cin, Cout), lambda n: (0, 0)),
            pl.BlockSpec((1, Cout), lambda n: (0, 0)),
        ],
        out_specs=pl.BlockSpec((nb, T, V, Cout), lambda n: (n, 0, 0, 0)),
        compiler_params=pltpu.CompilerParams(
            dimension_semantics=("parallel",)),
    )(X, A, w16, bias2)


# speedup vs baseline: 1.8276x; 1.2082x over previous
from functools import partial

import jax
import jax.numpy as jnp
from jax.experimental import pallas as pl
from jax.experimental.pallas import tpu as pltpu


def _gcn_body(x_ref, a_ref, w_ref, b_ref, o_ref, *, nb, T, V, Cout):
    bias = b_ref[...]
    for b in range(nb):
        a32 = a_ref[b]
        d = jnp.sum(a32, axis=-1, keepdims=True)
        dinv = jax.lax.rsqrt(d)
        a16 = a32.astype(jnp.bfloat16)

        x = x_ref[b].astype(jnp.bfloat16)
        s = jnp.dot(x, w_ref[...],
                    preferred_element_type=jnp.float32)
        s16 = (dinv * s).astype(jnp.bfloat16)

        agg = jnp.dot(a16, s16,
                      preferred_element_type=jnp.float32)

        for t in range(T):
            o_ref[b, t] = (dinv * agg[:, t * Cout:(t + 1) * Cout]
                           + bias).astype(o_ref.dtype)


def kernel(X, A, weight, bias):
    N, T, V, Cin = X.shape
    Cout = weight.shape[1]

    X2 = X.transpose(0, 2, 1, 3).reshape(N, V, T * Cin)
    W_bd = jnp.kron(jnp.eye(T, dtype=weight.dtype),
                    weight).astype(jnp.bfloat16)
    bias2 = bias.reshape(1, Cout)

    nb = next((c for c in (4, 2, 1) if N % c == 0), 1)
    G = N // nb

    return pl.pallas_call(
        partial(_gcn_body, nb=nb, T=T, V=V, Cout=Cout),
        out_shape=jax.ShapeDtypeStruct((N, T, V, Cout), X.dtype),
        grid=(G,),
        in_specs=[
            pl.BlockSpec((nb, V, T * Cin), lambda n: (n, 0, 0)),
            pl.BlockSpec((nb, V, V), lambda n: (n, 0, 0)),
            pl.BlockSpec((T * Cin, T * Cout), lambda n: (0, 0)),
            pl.BlockSpec((1, Cout), lambda n: (0, 0)),
        ],
        out_specs=pl.BlockSpec((nb, T, V, Cout), lambda n: (n, 0, 0, 0)),
        compiler_params=pltpu.CompilerParams(
            dimension_semantics=("parallel",)),
    )(X2, A, W_bd, bias2)


# speedup vs baseline: 1.8675x; 1.0218x over previous
from functools import partial

import jax
import jax.numpy as jnp
from jax.experimental import pallas as pl
from jax.experimental.pallas import tpu as pltpu


def _gcn_body(x_ref, a_ref, w_ref, b_ref, o_ref, *, nb, T, V, Cout):
    bias = b_ref[...]
    Cin = w_ref.shape[0]
    w = w_ref[...]
    for b in range(nb):
        a32 = a_ref[b]
        d = jnp.sum(a32, axis=-1, keepdims=True)
        dinv = jax.lax.rsqrt(d)
        a16 = a32.astype(jnp.bfloat16)

        xs = (dinv * x_ref[b]).astype(jnp.bfloat16)
        y = jnp.dot(a16, xs,
                    preferred_element_type=jnp.float32)

        for t in range(T):
            yt = y[:, t * Cin:(t + 1) * Cin].astype(jnp.bfloat16)
            proj = jnp.dot(yt, w,
                           preferred_element_type=jnp.float32)
            o_ref[b, t] = (dinv * proj + bias).astype(o_ref.dtype)


def kernel(X, A, weight, bias):
    N, T, V, Cin = X.shape
    Cout = weight.shape[1]

    X2 = X.transpose(0, 2, 1, 3).reshape(N, V, T * Cin)
    w16 = weight.astype(jnp.bfloat16)
    bias2 = bias.reshape(1, Cout)

    nb = next((c for c in (4, 2, 1) if N % c == 0), 1)
    G = N // nb

    return pl.pallas_call(
        partial(_gcn_body, nb=nb, T=T, V=V, Cout=Cout),
        out_shape=jax.ShapeDtypeStruct((N, T, V, Cout), X.dtype),
        grid=(G,),
        in_specs=[
            pl.BlockSpec((nb, V, T * Cin), lambda n: (n, 0, 0)),
            pl.BlockSpec((nb, V, V), lambda n: (n, 0, 0)),
            pl.BlockSpec((Cin, Cout), lambda n: (0, 0)),
            pl.BlockSpec((1, Cout), lambda n: (0, 0)),
        ],
        out_specs=pl.BlockSpec((nb, T, V, Cout), lambda n: (n, 0, 0, 0)),
        compiler_params=pltpu.CompilerParams(
            dimension_semantics=("parallel",)),
    )(X2, A, w16, bias2)
